# Initial kernel scaffold; baseline (speedup 1.0000x reference)
import jax
import jax.numpy as jnp
from jax.experimental import pallas as pl


def kernel(input_ids, token_type_ids, token_table, pos_table, seg_table, gamma, beta):
    raise NotImplementedError("write your pallas kernel here")


# speedup vs baseline: 1.1282x; 1.1282x over previous
import functools

import jax
import jax.numpy as jnp
from jax import lax
from jax.experimental import pallas as pl
from jax.experimental.pallas import tpu as pltpu
from jax.experimental.pallas import tpu_sc as plsc

_NC = 2
_NS = 16
_LANES = 16
_NW = _NC * _NS

_EPS = 1e-12


def _rsqrt(x):
    i = lax.bitcast_convert_type(x, jnp.int32)
    i = jnp.int32(0x5F3759DF) - lax.shift_right_arithmetic(i, 1)
    y = lax.bitcast_convert_type(i, jnp.float32)
    for _ in range(3):
        y = y * (1.5 - 0.5 * x * y * y)
    return y


def kernel(input_ids, token_type_ids, token_table, pos_table, seg_table, gamma, beta):
    B, L = input_ids.shape
    V, H = token_table.shape
    N = B * L
    NJ = H // _LANES

    C = 128
    per_w = N // _NW
    n_chunks = per_w // C

    ids = input_ids.reshape(N)
    tts = token_type_ids.reshape(N)

    mesh = plsc.VectorSubcoreMesh(
        core_axis_name="c", subcore_axis_name="s", num_cores=_NC, num_subcores=_NS
    )

    @functools.partial(
        pl.kernel,
        out_type=jax.ShapeDtypeStruct((N, H), jnp.float32),
        mesh=mesh,
        compiler_params=pltpu.CompilerParams(
            needs_layout_passes=False, use_tc_tiling_on_sc=False
        ),
        scratch_types=[
            pltpu.VMEM((L, H), jnp.float32),
            pltpu.VMEM((2, H), jnp.float32),
            pltpu.VMEM((H,), jnp.float32),
            pltpu.VMEM((H,), jnp.float32),
            pltpu.VMEM((C,), jnp.int32),
            pltpu.VMEM((C,), jnp.int32),
            pltpu.VMEM((C, H), jnp.float32),
            pltpu.SemaphoreType.DMA,
        ],
    )
    def sc_kernel(ids_hbm, tt_hbm, tab_hbm, pos_hbm, seg_hbm, g_hbm, b_hbm,
                  out_hbm, pos_v, seg_v, g_v, b_v, idx_v, tt_v, rows_v, sem):
        wid = lax.axis_index("s") * _NC + lax.axis_index("c")
        base = wid * per_w

        pltpu.sync_copy(pos_hbm, pos_v)
        pltpu.sync_copy(seg_hbm, seg_v)
        pltpu.sync_copy(g_hbm, g_v)
        pltpu.sync_copy(b_hbm, b_v)

        seg0 = [seg_v[0, pl.ds(j * _LANES, _LANES)] for j in range(NJ)]
        dseg = [seg_v[1, pl.ds(j * _LANES, _LANES)] - seg0[j] for j in range(NJ)]
        g_r = [g_v[pl.ds(j * _LANES, _LANES)] for j in range(NJ)]
        b_r = [b_v[pl.ds(j * _LANES, _LANES)] for j in range(NJ)]

        def chunk_body(c, carry):
            tok0 = base + c * C
            pltpu.sync_copy(ids_hbm.at[pl.ds(tok0, C)], idx_v)
            pltpu.sync_copy(tt_hbm.at[pl.ds(tok0, C)], tt_v)
            pltpu.async_copy(tab_hbm.at[idx_v], rows_v, sem).wait()

            def grp_body(g, carry2):
                ttv = tt_v[pl.ds(g * _LANES, _LANES)].astype(jnp.float32)
                for k in range(_LANES):
                    i = g * _LANES + k
                    lpos = lax.rem(c * C + i, L)
                    ttf = jnp.full((_LANES,), ttv[k], jnp.float32)
                    acc = []
                    for j in range(NJ):
                        sl = pl.ds(j * _LANES, _LANES)
                        acc.append(rows_v[i, sl] + pos_v[lpos, sl]
                                   + (seg0[j] + ttf * dseg[j]))
                    s = acc[0]
                    for j in range(1, NJ):
                        s = s + acc[j]
                    mean = jnp.sum(s) * (1.0 / H)
                    cen = [a - mean for a in acc]
                    sq = cen[0] * cen[0]
                    for j in range(1, NJ):
                        sq = sq + cen[j] * cen[j]
                    var = jnp.sum(sq) * (1.0 / H)
                    rstd = _rsqrt(var + _EPS)
                    for j in range(NJ):
                        rows_v[i, pl.ds(j * _LANES, _LANES)] = (
                            cen[j] * rstd * g_r[j] + b_r[j]
                        )
                return carry2

            lax.fori_loop(0, C // _LANES, grp_body, 0)
            pltpu.sync_copy(rows_v, out_hbm.at[pl.ds(tok0, C)])
            return carry

        lax.fori_loop(0, n_chunks, chunk_body, 0)

    out = sc_kernel(ids, tts, token_table, pos_table, seg_table, gamma, beta)
    return out.reshape(B, L, H)
